# baseline (speedup 1.0000x reference)
import functools

import jax
import jax.numpy as jnp
from jax import lax
from jax.experimental import pallas as pl
from jax.experimental.pallas import tpu as pltpu
from jax.experimental.pallas import tpu_sc as plsc

N = 10000
DIN = 128
H = 32
C = 10
NCLS = 7
E = 160000

NC = 2
NS = 16
NW = NC * NS
L = 16

NPAD = 10112
RPW = NPAD // NS
CSZ = 128
CH = 40
EPT = CH * CSZ
CH_LAST = (E - (NW - 1) * EPT) // CSZ
EP = NW * EPT
EPC = EP // CSZ

BL = 2000
GRID = N // BL

_MESH = dict(core_axis_name="c", subcore_axis_name="s", num_cores=NC,
             num_subcores=NS)
_SC_PARAMS = pltpu.CompilerParams(use_tc_tiling_on_sc=False)


def _nchunks(wid):
    return jnp.where(wid == NW - 1, CH_LAST, CH)


def _make_sc_deg():
    mesh = plsc.VectorSubcoreMesh(**_MESH)

    @functools.partial(
        pl.kernel,
        out_type=jax.ShapeDtypeStruct((NC * NPAD, 8), jnp.float32),
        mesh=mesh,
        compiler_params=_SC_PARAMS,
        scratch_types=[
            pltpu.VMEM((CH, CSZ), jnp.int32),
            pltpu.VMEM((CSZ, L), jnp.float32),
            pltpu.VMEM((RPW, L), jnp.float32),
            pltpu.VMEM_SHARED((NPAD, L), jnp.float32),
            pltpu.SemaphoreType.DMA,
        ],
    )
    def k(didx_hbm, out_hbm, si_v, ones_v, stage_v, acc_sh, sem):
        cid = lax.axis_index("c")
        sid = lax.axis_index("s")
        wid = sid * NC + cid
        pltpu.sync_copy(didx_hbm.at[pl.ds(wid * CH, CH)], si_v)
        one = jnp.full((L,), 1.0, jnp.float32)
        zero = jnp.zeros((L,), jnp.float32)

        def f1(i, _):
            ones_v[i, :] = one
            return 0

        lax.fori_loop(0, CSZ, f1, 0)

        def fz(i, _):
            stage_v[i, :] = zero
            return 0

        lax.fori_loop(0, RPW, fz, 0)
        pltpu.sync_copy(stage_v, acc_sh.at[pl.ds(sid * RPW, RPW)])
        plsc.subcore_barrier()
        nchk = _nchunks(wid)

        def fire(j, _):
            pltpu.async_copy(ones_v, acc_sh.at[si_v.at[j]], sem, add=True)
            return 0

        lax.fori_loop(0, nchk, fire, 0)

        def drain(j, _):
            pltpu.make_async_copy(ones_v, acc_sh.at[si_v.at[j]], sem).wait()
            return 0

        lax.fori_loop(0, nchk, drain, 0)
        plsc.subcore_barrier()
        pltpu.sync_copy(
            acc_sh.at[pl.ds(sid * RPW, RPW), pl.ds(0, 8)],
            out_hbm.at[pl.ds(cid * NPAD + sid * RPW, RPW)],
        )

    return k


def _make_sc_agg(W):
    mesh = plsc.VectorSubcoreMesh(**_MESH)
    NB = 4

    @functools.partial(
        pl.kernel,
        out_type=jax.ShapeDtypeStruct((NC * NPAD, W), jnp.float32),
        mesh=mesh,
        compiler_params=_SC_PARAMS,
        scratch_types=[
            pltpu.VMEM((CH, CSZ), jnp.int32),
            pltpu.VMEM((CH, CSZ), jnp.int32),
        ] + [pltpu.VMEM((CSZ, W), jnp.float32) for _ in range(NB)] + [
            pltpu.VMEM((RPW, W), jnp.float32),
            pltpu.VMEM_SHARED((NPAD, W), jnp.float32),
        ] + [pltpu.SemaphoreType.DMA for _ in range(2 * NB)],
    )
    def k(table_hbm, gidx_hbm, sidx_hbm, out_hbm, gi_v, si_v, r0, r1, r2, r3,
          stage_v, acc_sh, sg0, sg1, sg2, sg3, ss0, ss1, ss2, ss3):
        cid = lax.axis_index("c")
        sid = lax.axis_index("s")
        wid = sid * NC + cid
        bufs = (r0, r1, r2, r3)
        gsems = (sg0, sg1, sg2, sg3)
        ssems = (ss0, ss1, ss2, ss3)
        pltpu.sync_copy(gidx_hbm.at[pl.ds(wid * CH, CH)], gi_v)
        pltpu.sync_copy(sidx_hbm.at[pl.ds(wid * CH, CH)], si_v)
        zero = jnp.zeros((L,), jnp.float32)

        def fz(i, _):
            for w0 in range(W // L):
                stage_v[i, pl.ds(w0 * L, L)] = zero
            return 0

        lax.fori_loop(0, RPW, fz, 0)
        pltpu.sync_copy(stage_v, acc_sh.at[pl.ds(sid * RPW, RPW)])
        plsc.subcore_barrier()
        nchk = _nchunks(wid)

        def do_group(j, nb):
            gathers = [
                pltpu.async_copy(table_hbm.at[gi_v.at[j + b]], bufs[b],
                                 gsems[b])
                for b in range(nb)
            ]
            scatters = []
            for b in range(nb):
                gathers[b].wait()
                scatters.append(
                    pltpu.async_copy(bufs[b], acc_sh.at[si_v.at[j + b]],
                                     ssems[b], add=True))
            for b in range(nb):
                scatters[b].wait()

        def body(t, _):
            do_group(NB * t, NB)
            return 0

        nfull = nchk // NB
        lax.fori_loop(0, nfull, body, 0)

        @pl.when(nchk % NB != 0)
        def _():
            do_group(nfull * NB, 2)

        plsc.subcore_barrier()
        pltpu.sync_copy(
            acc_sh.at[pl.ds(sid * RPW, RPW)],
            out_hbm.at[pl.ds(cid * NPAD + sid * RPW, RPW)],
        )

    return k


def _make_sc_edge():
    mesh = plsc.VectorSubcoreMesh(**_MESH)

    @functools.partial(
        pl.kernel,
        out_type=jax.ShapeDtypeStruct((NW * 2, L), jnp.float32),
        mesh=mesh,
        compiler_params=_SC_PARAMS,
        scratch_types=[
            pltpu.VMEM((CH, CSZ), jnp.int32),
            pltpu.VMEM((CH, CSZ), jnp.int32),
            pltpu.VMEM((CSZ, L), jnp.float32),
            pltpu.VMEM((CSZ, L), jnp.float32),
            pltpu.VMEM((CSZ, L), jnp.float32),
            pltpu.VMEM((CSZ, L), jnp.float32),
            pltpu.VMEM((2, L), jnp.float32),
            pltpu.SemaphoreType.DMA,
            pltpu.SemaphoreType.DMA,
        ],
    )
    def k(st_hbm, gidx_hbm, sidx_hbm, out_hbm, si_v, di_v, ra0, rb0, ra1, rb1,
          res_v, sem0, sem1):
        cid = lax.axis_index("c")
        sid = lax.axis_index("s")
        wid = sid * NC + cid
        pltpu.sync_copy(gidx_hbm.at[pl.ds(wid * CH, CH)], si_v)
        pltpu.sync_copy(sidx_hbm.at[pl.ds(wid * CH, CH)], di_v)
        zero = jnp.zeros((L,), jnp.float32)

        def accum(ra, rb, carry):
            def inner(r, c2):
                accs = list(c2)
                base = r * 8
                for u in range(8):
                    va = ra[base + u]
                    vb = rb[base + u]
                    p = u % 4
                    accs[p] = accs[p] + va * vb
                    accs[4 + p] = accs[4 + p] + va * va
                return tuple(accs)

            return lax.fori_loop(0, CSZ // 8, inner, carry)

        def body(t, carry):
            j0 = 2 * t
            j1 = j0 + 1
            cpa0 = pltpu.async_copy(st_hbm.at[si_v.at[j0]], ra0, sem0)
            cpb0 = pltpu.async_copy(st_hbm.at[di_v.at[j0]], rb0, sem0)
            cpa1 = pltpu.async_copy(st_hbm.at[si_v.at[j1]], ra1, sem1)
            cpb1 = pltpu.async_copy(st_hbm.at[di_v.at[j1]], rb1, sem1)
            cpa0.wait()
            cpb0.wait()
            carry = accum(ra0, rb0, carry)
            cpa1.wait()
            cpb1.wait()
            return accum(ra1, rb1, carry)

        init = tuple(zero for _ in range(8))
        accs = lax.fori_loop(0, _nchunks(wid) // 2, body, init)
        res_v[0] = accs[0] + accs[1] + accs[2] + accs[3]
        res_v[1] = accs[4] + accs[5] + accs[6] + accs[7]
        pltpu.sync_copy(res_v, out_hbm.at[pl.ds(wid * 2, 2)])

    return k


_sc_deg = _make_sc_deg()
_sc_agg = _make_sc_agg(H)
_sc_edge = _make_sc_edge()


def _tc_b_body(x_ref, w1_ref, dp_ref, u1_ref, dis_ref):
    deg = dp_ref[0] + dp_ref[1] + 1.0
    dis = lax.rsqrt(deg)[:, 0:1]
    h1 = jnp.dot(x_ref[...], w1_ref[...], preferred_element_type=jnp.float32)
    u1_ref[...] = dis * h1
    dis_ref[...] = dis


def _tc_b(x, w1, degp):
    return pl.pallas_call(
        _tc_b_body,
        grid=(GRID,),
        in_specs=[
            pl.BlockSpec((BL, DIN), lambda i: (i, 0)),
            pl.BlockSpec((DIN, H), lambda i: (0, 0)),
            pl.BlockSpec((NC, BL, 8), lambda i: (0, i, 0)),
        ],
        out_specs=[
            pl.BlockSpec((BL, H), lambda i: (i, 0)),
            pl.BlockSpec((BL, 1), lambda i: (i, 0)),
        ],
        out_shape=[
            jax.ShapeDtypeStruct((N, H), jnp.float32),
            jax.ShapeDtypeStruct((N, 1), jnp.float32),
        ],
    )(x, w1, degp)


def _tc_c_body(ap_ref, u1_ref, dis_ref, b1_ref, w2_ref, u2_ref):
    d0 = dis_ref[...]
    h = jnp.maximum(
        d0 * (ap_ref[0] + ap_ref[1] + u1_ref[...]) + b1_ref[...], 0.0)
    u2_ref[...] = d0 * jnp.dot(h, w2_ref[...],
                               preferred_element_type=jnp.float32)


def _tc_c(ap, u1, dis, b1r, w2):
    return pl.pallas_call(
        _tc_c_body,
        grid=(GRID,),
        in_specs=[
            pl.BlockSpec((NC, BL, H), lambda i: (0, i, 0)),
            pl.BlockSpec((BL, H), lambda i: (i, 0)),
            pl.BlockSpec((BL, 1), lambda i: (i, 0)),
            pl.BlockSpec((1, H), lambda i: (0, 0)),
            pl.BlockSpec((H, H), lambda i: (0, 0)),
        ],
        out_specs=pl.BlockSpec((BL, H), lambda i: (i, 0)),
        out_shape=jax.ShapeDtypeStruct((N, H), jnp.float32),
    )(ap, u1, dis, b1r, w2)


def _tc_d_body(ap_ref, u2_ref, dis_ref, b2_ref, wp_ref, bp_ref,
               wc_ref, bc_ref, st_ref, ss_ref, lp_ref):
    i = pl.program_id(0)
    d0 = dis_ref[...]
    h = jnp.maximum(
        d0 * (ap_ref[0] + ap_ref[1] + u2_ref[...]) + b2_ref[...], 0.0)
    sl = jnp.dot(h, wp_ref[...], preferred_element_type=jnp.float32) \
        + bp_ref[...]
    m = jnp.max(sl, axis=1, keepdims=True)
    p = jnp.exp(sl - m)
    s = p / jnp.sum(p, axis=1, keepdims=True)
    st_ref[...] = jnp.concatenate(
        [s, jnp.ones((BL, 1), jnp.float32), jnp.zeros((BL, 5), jnp.float32)],
        axis=1)
    ssb = lax.dot_general(s, s, (((0,), (0,)), ((), ())),
                          preferred_element_type=jnp.float32)

    @pl.when(i == 0)
    def _():
        ss_ref[...] = jnp.zeros_like(ss_ref)

    ss_ref[...] += ssb
    lo = jnp.dot(h, wc_ref[...], preferred_element_type=jnp.float32) \
        + bc_ref[...]
    mm = jnp.max(lo, axis=1, keepdims=True)
    lp_ref[...] = lo - mm - jnp.log(
        jnp.sum(jnp.exp(lo - mm), axis=1, keepdims=True))


def _tc_d(ap, u2, dis, b2r, wp, bpr, wc, bcr):
    return pl.pallas_call(
        _tc_d_body,
        grid=(GRID,),
        in_specs=[
            pl.BlockSpec((NC, BL, H), lambda i: (0, i, 0)),
            pl.BlockSpec((BL, H), lambda i: (i, 0)),
            pl.BlockSpec((BL, 1), lambda i: (i, 0)),
            pl.BlockSpec((1, H), lambda i: (0, 0)),
            pl.BlockSpec((H, C), lambda i: (0, 0)),
            pl.BlockSpec((1, C), lambda i: (0, 0)),
            pl.BlockSpec((H, NCLS), lambda i: (0, 0)),
            pl.BlockSpec((1, NCLS), lambda i: (0, 0)),
        ],
        out_specs=[
            pl.BlockSpec((BL, L), lambda i: (i, 0)),
            pl.BlockSpec((C, C), lambda i: (0, 0)),
            pl.BlockSpec((BL, NCLS), lambda i: (i, 0)),
        ],
        out_shape=[
            jax.ShapeDtypeStruct((N, L), jnp.float32),
            jax.ShapeDtypeStruct((C, C), jnp.float32),
            jax.ShapeDtypeStruct((N, NCLS), jnp.float32),
        ],
    )(ap, u2, dis, b2r, wp, bpr, wc, bcr)


def _tc_f_body(na_ref, da_ref, ss_ref, mc_ref, o_ref):
    num = jnp.sum(na_ref[...]) - float(E)
    den = jnp.sum(da_ref[...]) - float(E)
    mc_ref[...] = jnp.full((1, 1), -(num / den), jnp.float32)
    ssv = ss_ref[...]
    nss = jnp.sqrt(jnp.sum(ssv * ssv))
    eye = (lax.broadcasted_iota(jnp.int32, (C, C), 0)
           == lax.broadcasted_iota(jnp.int32, (C, C), 1)).astype(jnp.float32)
    t = ssv / nss - eye / jnp.sqrt(jnp.float32(C))
    o_ref[...] = jnp.full((1, 1), jnp.sqrt(jnp.sum(t * t)), jnp.float32)


def _tc_f(na, da, ss):
    return pl.pallas_call(
        _tc_f_body,
        out_shape=[
            jax.ShapeDtypeStruct((1, 1), jnp.float32),
            jax.ShapeDtypeStruct((1, 1), jnp.float32),
        ],
    )(na, da, ss)


def kernel(x, edge_index, W1, b1, W2, b2, Wp, bp, Wc, bc):
    pad = jnp.zeros((EP - E,), jnp.int32)
    srcp = jnp.concatenate([edge_index[0], pad]).reshape(EPC, CSZ)
    dstp = jnp.concatenate([edge_index[1], pad]).reshape(EPC, CSZ)

    degp = _sc_deg(dstp).reshape(NC, NPAD, 8)
    u1, dis = _tc_b(x, W1, degp)

    a1 = _sc_agg(u1, srcp, dstp).reshape(NC, NPAD, H)
    u2 = _tc_c(a1, u1, dis, b1.reshape(1, H), W2)

    a2 = _sc_agg(u2, srcp, dstp).reshape(NC, NPAD, H)
    st, ss, logp = _tc_d(a2, u2, dis, b2.reshape(1, H),
                         Wp, bp.reshape(1, C), Wc, bc.reshape(1, NCLS))

    eacc = _sc_edge(st, srcp, dstp).reshape(NW, 2, L)
    mc, o = _tc_f(eacc[:, 0], eacc[:, 1], ss)
    return (logp, jnp.reshape(mc, ()), jnp.reshape(o, ()))


# speedup vs baseline: 1.0583x; 1.0583x over previous
import functools

import jax
import jax.numpy as jnp
from jax import lax
from jax.experimental import pallas as pl
from jax.experimental.pallas import tpu as pltpu
from jax.experimental.pallas import tpu_sc as plsc

N = 10000
DIN = 128
H = 32
C = 10
NCLS = 7
E = 160000

NC = 2
NS = 16
NW = NC * NS
L = 16

NPAD = 10112
RPW = NPAD // NS
CSZ = 128
CH = 40
EPT = CH * CSZ
CH_LAST = (E - (NW - 1) * EPT) // CSZ
EP = NW * EPT
EPC = EP // CSZ

BL = 2000
GRID = N // BL

_MESH = dict(core_axis_name="c", subcore_axis_name="s", num_cores=NC,
             num_subcores=NS)
_SC_PARAMS = pltpu.CompilerParams(use_tc_tiling_on_sc=False)


def _nchunks(wid):
    return jnp.where(wid == NW - 1, CH_LAST, CH)


def _make_sc_deg():
    mesh = plsc.VectorSubcoreMesh(**_MESH)

    @functools.partial(
        pl.kernel,
        out_type=jax.ShapeDtypeStruct((NC * NPAD, L), jnp.float32),
        mesh=mesh,
        compiler_params=_SC_PARAMS,
        scratch_types=[
            pltpu.VMEM((CH, CSZ), jnp.int32),
            pltpu.VMEM((CSZ, L), jnp.float32),
            pltpu.VMEM((RPW, L), jnp.float32),
            pltpu.VMEM_SHARED((NPAD, L), jnp.float32),
            pltpu.SemaphoreType.DMA,
        ],
    )
    def k(ei_hbm, out_hbm, si_v, ones_v, stage_v, acc_sh, sem):
        cid = lax.axis_index("c")
        sid = lax.axis_index("s")
        wid = sid * NC + cid
        pltpu.sync_copy(ei_hbm.at[1, pl.ds(wid * CH, CH)], si_v)
        one = jnp.full((L,), 1.0, jnp.float32)
        zero = jnp.zeros((L,), jnp.float32)

        def f1(i, _):
            ones_v[i, :] = one
            return 0

        lax.fori_loop(0, CSZ, f1, 0)

        def fz(i, _):
            stage_v[i, :] = zero
            return 0

        lax.fori_loop(0, RPW, fz, 0)
        pltpu.sync_copy(stage_v, acc_sh.at[pl.ds(sid * RPW, RPW)])
        plsc.subcore_barrier()
        nchk = _nchunks(wid)

        def fire(j, _):
            pltpu.async_copy(ones_v, acc_sh.at[si_v.at[j]], sem, add=True)
            return 0

        lax.fori_loop(0, nchk, fire, 0)

        def drain(j, _):
            pltpu.make_async_copy(ones_v, acc_sh.at[si_v.at[j]], sem).wait()
            return 0

        lax.fori_loop(0, nchk, drain, 0)
        plsc.subcore_barrier()
        pltpu.sync_copy(
            acc_sh.at[pl.ds(sid * RPW, RPW)],
            out_hbm.at[pl.ds(cid * NPAD + sid * RPW, RPW)],
        )

    return k


def _make_sc_agg(W):
    mesh = plsc.VectorSubcoreMesh(**_MESH)
    NB = 4

    @functools.partial(
        pl.kernel,
        out_type=jax.ShapeDtypeStruct((NC * NPAD, W), jnp.float32),
        mesh=mesh,
        compiler_params=_SC_PARAMS,
        scratch_types=[
            pltpu.VMEM((CH, CSZ), jnp.int32),
            pltpu.VMEM((CH, CSZ), jnp.int32),
        ] + [pltpu.VMEM((CSZ, W), jnp.float32) for _ in range(NB)] + [
            pltpu.VMEM((RPW, W), jnp.float32),
            pltpu.VMEM_SHARED((NPAD, W), jnp.float32),
        ] + [pltpu.SemaphoreType.DMA for _ in range(2 * NB)],
    )
    def k(table_hbm, ei_hbm, out_hbm, gi_v, si_v, r0, r1, r2, r3,
          stage_v, acc_sh, sg0, sg1, sg2, sg3, ss0, ss1, ss2, ss3):
        cid = lax.axis_index("c")
        sid = lax.axis_index("s")
        wid = sid * NC + cid
        bufs = (r0, r1, r2, r3)
        gsems = (sg0, sg1, sg2, sg3)
        ssems = (ss0, ss1, ss2, ss3)
        pltpu.sync_copy(ei_hbm.at[0, pl.ds(wid * CH, CH)], gi_v)
        pltpu.sync_copy(ei_hbm.at[1, pl.ds(wid * CH, CH)], si_v)
        zero = jnp.zeros((L,), jnp.float32)

        def fz(i, _):
            for w0 in range(W // L):
                stage_v[i, pl.ds(w0 * L, L)] = zero
            return 0

        lax.fori_loop(0, RPW, fz, 0)
        pltpu.sync_copy(stage_v, acc_sh.at[pl.ds(sid * RPW, RPW)])
        plsc.subcore_barrier()
        nchk = _nchunks(wid)

        def do_group(j, nb):
            gathers = [
                pltpu.async_copy(table_hbm.at[gi_v.at[j + b]], bufs[b],
                                 gsems[b])
                for b in range(nb)
            ]
            scatters = []
            for b in range(nb):
                gathers[b].wait()
                scatters.append(
                    pltpu.async_copy(bufs[b], acc_sh.at[si_v.at[j + b]],
                                     ssems[b], add=True))
            for b in range(nb):
                scatters[b].wait()

        def body(t, _):
            do_group(NB * t, NB)
            return 0

        nfull = nchk // NB
        lax.fori_loop(0, nfull, body, 0)

        @pl.when(nchk % NB != 0)
        def _():
            do_group(nfull * NB, 2)

        plsc.subcore_barrier()
        pltpu.sync_copy(
            acc_sh.at[pl.ds(sid * RPW, RPW)],
            out_hbm.at[pl.ds(cid * NPAD + sid * RPW, RPW)],
        )

    return k


def _make_sc_edge():
    mesh = plsc.VectorSubcoreMesh(**_MESH)

    @functools.partial(
        pl.kernel,
        out_type=jax.ShapeDtypeStruct((NW * 2, L), jnp.float32),
        mesh=mesh,
        compiler_params=_SC_PARAMS,
        scratch_types=[
            pltpu.VMEM((CH, CSZ), jnp.int32),
            pltpu.VMEM((CH, CSZ), jnp.int32),
            pltpu.VMEM((CSZ, L), jnp.float32),
            pltpu.VMEM((CSZ, L), jnp.float32),
            pltpu.VMEM((CSZ, L), jnp.float32),
            pltpu.VMEM((CSZ, L), jnp.float32),
            pltpu.VMEM((2, L), jnp.float32),
            pltpu.SemaphoreType.DMA,
            pltpu.SemaphoreType.DMA,
        ],
    )
    def k(st_hbm, ei_hbm, out_hbm, si_v, di_v, ra0, rb0, ra1, rb1,
          res_v, sem0, sem1):
        cid = lax.axis_index("c")
        sid = lax.axis_index("s")
        wid = sid * NC + cid
        pltpu.sync_copy(ei_hbm.at[0, pl.ds(wid * CH, CH)], si_v)
        pltpu.sync_copy(ei_hbm.at[1, pl.ds(wid * CH, CH)], di_v)
        zero = jnp.zeros((L,), jnp.float32)

        def accum(ra, rb, carry):
            def inner(r, c2):
                accs = list(c2)
                base = r * 8
                for u in range(8):
                    va = ra[base + u]
                    vb = rb[base + u]
                    p = u % 4
                    accs[p] = accs[p] + va * vb
                    accs[4 + p] = accs[4 + p] + va * va
                return tuple(accs)

            return lax.fori_loop(0, CSZ // 8, inner, carry)

        def body(t, carry):
            j0 = 2 * t
            j1 = j0 + 1
            cpa0 = pltpu.async_copy(st_hbm.at[si_v.at[j0]], ra0, sem0)
            cpb0 = pltpu.async_copy(st_hbm.at[di_v.at[j0]], rb0, sem0)
            cpa1 = pltpu.async_copy(st_hbm.at[si_v.at[j1]], ra1, sem1)
            cpb1 = pltpu.async_copy(st_hbm.at[di_v.at[j1]], rb1, sem1)
            cpa0.wait()
            cpb0.wait()
            carry = accum(ra0, rb0, carry)
            cpa1.wait()
            cpb1.wait()
            return accum(ra1, rb1, carry)

        init = tuple(zero for _ in range(8))
        accs = lax.fori_loop(0, _nchunks(wid) // 2, body, init)
        res_v[0] = accs[0] + accs[1] + accs[2] + accs[3]
        res_v[1] = accs[4] + accs[5] + accs[6] + accs[7]
        pltpu.sync_copy(res_v, out_hbm.at[pl.ds(wid * 2, 2)])

    return k


_sc_deg = _make_sc_deg()
_sc_agg = _make_sc_agg(H)
_sc_edge = _make_sc_edge()


def _tc_a_body(x_ref, w1_ref, h1_ref):
    h1_ref[...] = jnp.dot(x_ref[...], w1_ref[...],
                          preferred_element_type=jnp.float32)


def _tc_a(x, w1):
    return pl.pallas_call(
        _tc_a_body,
        grid=(GRID,),
        in_specs=[
            pl.BlockSpec((BL, DIN), lambda i: (i, 0)),
            pl.BlockSpec((DIN, H), lambda i: (0, 0)),
        ],
        out_specs=pl.BlockSpec((BL, H), lambda i: (i, 0)),
        out_shape=jax.ShapeDtypeStruct((N, H), jnp.float32),
    )(x, w1)


def _tc_b_body(h1_ref, dp_ref, u1_ref, dis_ref):
    deg = dp_ref[0] + dp_ref[1] + 1.0
    dis = lax.rsqrt(deg)[:, 0:1]
    u1_ref[...] = dis * h1_ref[...]
    dis_ref[...] = dis


def _tc_b(h1, degp):
    return pl.pallas_call(
        _tc_b_body,
        grid=(GRID,),
        in_specs=[
            pl.BlockSpec((BL, H), lambda i: (i, 0)),
            pl.BlockSpec((NC, BL, L), lambda i: (0, i, 0)),
        ],
        out_specs=[
            pl.BlockSpec((BL, H), lambda i: (i, 0)),
            pl.BlockSpec((BL, 1), lambda i: (i, 0)),
        ],
        out_shape=[
            jax.ShapeDtypeStruct((N, H), jnp.float32),
            jax.ShapeDtypeStruct((N, 1), jnp.float32),
        ],
    )(h1, degp)


def _tc_c_body(ap_ref, u1_ref, dis_ref, b1_ref, w2_ref, u2_ref):
    d0 = dis_ref[...]
    h = jnp.maximum(
        d0 * (ap_ref[0] + ap_ref[1] + u1_ref[...]) + b1_ref[...], 0.0)
    u2_ref[...] = d0 * jnp.dot(h, w2_ref[...],
                               preferred_element_type=jnp.float32)


def _tc_c(ap, u1, dis, b1r, w2):
    return pl.pallas_call(
        _tc_c_body,
        grid=(GRID,),
        in_specs=[
            pl.BlockSpec((NC, BL, H), lambda i: (0, i, 0)),
            pl.BlockSpec((BL, H), lambda i: (i, 0)),
            pl.BlockSpec((BL, 1), lambda i: (i, 0)),
            pl.BlockSpec((1, H), lambda i: (0, 0)),
            pl.BlockSpec((H, H), lambda i: (0, 0)),
        ],
        out_specs=pl.BlockSpec((BL, H), lambda i: (i, 0)),
        out_shape=jax.ShapeDtypeStruct((N, H), jnp.float32),
    )(ap, u1, dis, b1r, w2)


def _tc_d_body(ap_ref, u2_ref, dis_ref, b2_ref, wp_ref, bp_ref,
               st_ref, ss_ref, h_ref):
    i = pl.program_id(0)
    d0 = dis_ref[...]
    h = jnp.maximum(
        d0 * (ap_ref[0] + ap_ref[1] + u2_ref[...]) + b2_ref[...], 0.0)
    h_ref[...] = h
    sl = jnp.dot(h, wp_ref[...], preferred_element_type=jnp.float32) \
        + bp_ref[...]
    m = jnp.max(sl, axis=1, keepdims=True)
    p = jnp.exp(sl - m)
    s = p / jnp.sum(p, axis=1, keepdims=True)
    st_ref[...] = jnp.concatenate(
        [s, jnp.ones((BL, 1), jnp.float32), jnp.zeros((BL, 5), jnp.float32)],
        axis=1)
    ssb = lax.dot_general(s, s, (((0,), (0,)), ((), ())),
                          preferred_element_type=jnp.float32)

    @pl.when(i == 0)
    def _():
        ss_ref[...] = jnp.zeros_like(ss_ref)

    ss_ref[...] += ssb


def _tc_d(ap, u2, dis, b2r, wp, bpr):
    return pl.pallas_call(
        _tc_d_body,
        grid=(GRID,),
        in_specs=[
            pl.BlockSpec((NC, BL, H), lambda i: (0, i, 0)),
            pl.BlockSpec((BL, H), lambda i: (i, 0)),
            pl.BlockSpec((BL, 1), lambda i: (i, 0)),
            pl.BlockSpec((1, H), lambda i: (0, 0)),
            pl.BlockSpec((H, C), lambda i: (0, 0)),
            pl.BlockSpec((1, C), lambda i: (0, 0)),
        ],
        out_specs=[
            pl.BlockSpec((BL, L), lambda i: (i, 0)),
            pl.BlockSpec((C, C), lambda i: (0, 0)),
            pl.BlockSpec((BL, H), lambda i: (i, 0)),
        ],
        out_shape=[
            jax.ShapeDtypeStruct((N, L), jnp.float32),
            jax.ShapeDtypeStruct((C, C), jnp.float32),
            jax.ShapeDtypeStruct((N, H), jnp.float32),
        ],
    )(ap, u2, dis, b2r, wp, bpr)


def _tc_d2_body(h_ref, wc_ref, bc_ref, lp_ref):
    lo = jnp.dot(h_ref[...], wc_ref[...],
                 preferred_element_type=jnp.float32) + bc_ref[...]
    mm = jnp.max(lo, axis=1, keepdims=True)
    lp_ref[...] = lo - mm - jnp.log(
        jnp.sum(jnp.exp(lo - mm), axis=1, keepdims=True))


def _tc_d2(h, wc, bcr):
    return pl.pallas_call(
        _tc_d2_body,
        grid=(GRID,),
        in_specs=[
            pl.BlockSpec((BL, H), lambda i: (i, 0)),
            pl.BlockSpec((H, NCLS), lambda i: (0, 0)),
            pl.BlockSpec((1, NCLS), lambda i: (0, 0)),
        ],
        out_specs=pl.BlockSpec((BL, NCLS), lambda i: (i, 0)),
        out_shape=jax.ShapeDtypeStruct((N, NCLS), jnp.float32),
    )(h, wc, bcr)


def _tc_f_body(na_ref, da_ref, ss_ref, mc_ref, o_ref):
    num = jnp.sum(na_ref[...]) - float(E)
    den = jnp.sum(da_ref[...]) - float(E)
    mc_ref[...] = jnp.full((1, 1), -(num / den), jnp.float32)
    ssv = ss_ref[...]
    nss = jnp.sqrt(jnp.sum(ssv * ssv))
    eye = (lax.broadcasted_iota(jnp.int32, (C, C), 0)
           == lax.broadcasted_iota(jnp.int32, (C, C), 1)).astype(jnp.float32)
    t = ssv / nss - eye / jnp.sqrt(jnp.float32(C))
    o_ref[...] = jnp.full((1, 1), jnp.sqrt(jnp.sum(t * t)), jnp.float32)


def _tc_f(na, da, ss):
    return pl.pallas_call(
        _tc_f_body,
        out_shape=[
            jax.ShapeDtypeStruct((1, 1), jnp.float32),
            jax.ShapeDtypeStruct((1, 1), jnp.float32),
        ],
    )(na, da, ss)


def kernel(x, edge_index, W1, b1, W2, b2, Wp, bp, Wc, bc):
    ei = jnp.pad(edge_index, ((0, 0), (0, EP - E))).reshape(2, EPC, CSZ)

    degp = _sc_deg(ei).reshape(NC, NPAD, L)
    h1 = _tc_a(x, W1)
    u1, dis = _tc_b(h1, degp)

    a1 = _sc_agg(u1, ei).reshape(NC, NPAD, H)
    u2 = _tc_c(a1, u1, dis, b1.reshape(1, H), W2)

    a2 = _sc_agg(u2, ei).reshape(NC, NPAD, H)
    st, ss, h = _tc_d(a2, u2, dis, b2.reshape(1, H), Wp, bp.reshape(1, C))

    eacc = _sc_edge(st, ei).reshape(NW, 2, L)
    logp = _tc_d2(h, Wc, bc.reshape(1, NCLS))
    mc, o = _tc_f(eacc[:, 0], eacc[:, 1], ss)
    return (logp, jnp.reshape(mc, ()), jnp.reshape(o, ()))


# speedup vs baseline: 1.0847x; 1.0249x over previous
import functools

import jax
import jax.numpy as jnp
from jax import lax
from jax.experimental import pallas as pl
from jax.experimental.pallas import tpu as pltpu
from jax.experimental.pallas import tpu_sc as plsc

N = 10000
DIN = 128
H = 32
C = 10
NCLS = 7
E = 160000

NC = 2
NS = 16
NW = NC * NS
L = 16

NPAD = 10112
RPW = NPAD // NS
CSZ = 128
CH = 40
EPT = CH * CSZ
CH_LAST = (E - (NW - 1) * EPT) // CSZ
EP = NW * EPT
EPC = EP // CSZ

BL = 2000
GRID = N // BL

_MESH = dict(core_axis_name="c", subcore_axis_name="s", num_cores=NC,
             num_subcores=NS)
_SC_PARAMS = pltpu.CompilerParams(use_tc_tiling_on_sc=False)


def _nchunks(wid):
    return jnp.where(wid == NW - 1, CH_LAST, CH)


def _make_sc_deg():
    mesh = plsc.VectorSubcoreMesh(**_MESH)

    @functools.partial(
        pl.kernel,
        out_type=jax.ShapeDtypeStruct((NC * NPAD, L), jnp.float32),
        mesh=mesh,
        compiler_params=_SC_PARAMS,
        scratch_types=[
            pltpu.VMEM((CH, CSZ), jnp.int32),
            pltpu.VMEM((CSZ, L), jnp.float32),
            pltpu.VMEM((RPW, L), jnp.float32),
            pltpu.VMEM_SHARED((NPAD, L), jnp.float32),
            pltpu.SemaphoreType.DMA,
        ],
    )
    def k(ei_hbm, out_hbm, si_v, ones_v, stage_v, acc_sh, sem):
        cid = lax.axis_index("c")
        sid = lax.axis_index("s")
        wid = sid * NC + cid
        pltpu.sync_copy(ei_hbm.at[1, pl.ds(wid * CH, CH)], si_v)
        one = jnp.full((L,), 1.0, jnp.float32)
        zero = jnp.zeros((L,), jnp.float32)

        def f1(i, _):
            ones_v[i, :] = one
            return 0

        lax.fori_loop(0, CSZ, f1, 0)

        def fz(i, _):
            stage_v[i, :] = zero
            return 0

        lax.fori_loop(0, RPW, fz, 0)
        pltpu.sync_copy(stage_v, acc_sh.at[pl.ds(sid * RPW, RPW)])
        plsc.subcore_barrier()
        nchk = _nchunks(wid)

        def fire(j, _):
            pltpu.async_copy(ones_v, acc_sh.at[si_v.at[j]], sem, add=True)
            return 0

        lax.fori_loop(0, nchk, fire, 0)

        def drain(j, _):
            pltpu.make_async_copy(ones_v, acc_sh.at[si_v.at[j]], sem).wait()
            return 0

        lax.fori_loop(0, nchk, drain, 0)
        plsc.subcore_barrier()
        pltpu.sync_copy(
            acc_sh.at[pl.ds(sid * RPW, RPW)],
            out_hbm.at[pl.ds(cid * NPAD + sid * RPW, RPW)],
        )

    return k


def _make_sc_agg(W):
    mesh = plsc.VectorSubcoreMesh(**_MESH)
    NB = 4

    @functools.partial(
        pl.kernel,
        out_type=jax.ShapeDtypeStruct((NC * NPAD, W), jnp.float32),
        mesh=mesh,
        compiler_params=_SC_PARAMS,
        scratch_types=[
            pltpu.VMEM((CH, CSZ), jnp.int32),
            pltpu.VMEM((CH, CSZ), jnp.int32),
        ] + [pltpu.VMEM((CSZ, W), jnp.float32) for _ in range(NB)] + [
            pltpu.VMEM((RPW, W), jnp.float32),
            pltpu.VMEM_SHARED((NPAD, W), jnp.float32),
            pltpu.VMEM_SHARED((N, W), jnp.float32),
        ] + [pltpu.SemaphoreType.DMA for _ in range(2 * NB)],
    )
    def k(table_hbm, ei_hbm, out_hbm, gi_v, si_v, r0, r1, r2, r3,
          stage_v, acc_sh, tab_sh, sg0, sg1, sg2, sg3, ss0, ss1, ss2, ss3):
        cid = lax.axis_index("c")
        sid = lax.axis_index("s")
        wid = sid * NC + cid
        bufs = (r0, r1, r2, r3)
        gsems = (sg0, sg1, sg2, sg3)
        ssems = (ss0, ss1, ss2, ss3)
        TRW = 624
        pltpu.sync_copy(table_hbm.at[pl.ds(sid * TRW, TRW)],
                        tab_sh.at[pl.ds(sid * TRW, TRW)])

        @pl.when(sid == NS - 1)
        def _():
            pltpu.sync_copy(table_hbm.at[pl.ds(NS * TRW, N - NS * TRW)],
                            tab_sh.at[pl.ds(NS * TRW, N - NS * TRW)])

        pltpu.sync_copy(ei_hbm.at[0, pl.ds(wid * CH, CH)], gi_v)
        pltpu.sync_copy(ei_hbm.at[1, pl.ds(wid * CH, CH)], si_v)
        zero = jnp.zeros((L,), jnp.float32)

        def fz(i, _):
            for w0 in range(W // L):
                stage_v[i, pl.ds(w0 * L, L)] = zero
            return 0

        lax.fori_loop(0, RPW, fz, 0)
        pltpu.sync_copy(stage_v, acc_sh.at[pl.ds(sid * RPW, RPW)])
        plsc.subcore_barrier()
        nchk = _nchunks(wid)

        def do_group(j, nb):
            gathers = [
                pltpu.async_copy(tab_sh.at[gi_v.at[j + b]], bufs[b],
                                 gsems[b])
                for b in range(nb)
            ]
            scatters = []
            for b in range(nb):
                gathers[b].wait()
                scatters.append(
                    pltpu.async_copy(bufs[b], acc_sh.at[si_v.at[j + b]],
                                     ssems[b], add=True))
            for b in range(nb):
                scatters[b].wait()

        def body(t, _):
            do_group(NB * t, NB)
            return 0

        nfull = nchk // NB
        lax.fori_loop(0, nfull, body, 0)

        @pl.when(nchk % NB != 0)
        def _():
            do_group(nfull * NB, 2)

        plsc.subcore_barrier()
        pltpu.sync_copy(
            acc_sh.at[pl.ds(sid * RPW, RPW)],
            out_hbm.at[pl.ds(cid * NPAD + sid * RPW, RPW)],
        )

    return k


def _make_sc_edge():
    mesh = plsc.VectorSubcoreMesh(**_MESH)

    @functools.partial(
        pl.kernel,
        out_type=jax.ShapeDtypeStruct((NW * 2, L), jnp.float32),
        mesh=mesh,
        compiler_params=_SC_PARAMS,
        scratch_types=[
            pltpu.VMEM((CH, CSZ), jnp.int32),
            pltpu.VMEM((CH, CSZ), jnp.int32),
            pltpu.VMEM((CSZ, L), jnp.float32),
            pltpu.VMEM((CSZ, L), jnp.float32),
            pltpu.VMEM((CSZ, L), jnp.float32),
            pltpu.VMEM((CSZ, L), jnp.float32),
            pltpu.VMEM((2, L), jnp.float32),
            pltpu.SemaphoreType.DMA,
            pltpu.SemaphoreType.DMA,
        ],
    )
    def k(st_hbm, ei_hbm, out_hbm, si_v, di_v, ra0, rb0, ra1, rb1,
          res_v, sem0, sem1):
        cid = lax.axis_index("c")
        sid = lax.axis_index("s")
        wid = sid * NC + cid
        pltpu.sync_copy(ei_hbm.at[0, pl.ds(wid * CH, CH)], si_v)
        pltpu.sync_copy(ei_hbm.at[1, pl.ds(wid * CH, CH)], di_v)
        zero = jnp.zeros((L,), jnp.float32)

        def accum(ra, rb, carry):
            def inner(r, c2):
                accs = list(c2)
                base = r * 8
                for u in range(8):
                    va = ra[base + u]
                    vb = rb[base + u]
                    p = u % 4
                    accs[p] = accs[p] + va * vb
                    accs[4 + p] = accs[4 + p] + va * va
                return tuple(accs)

            return lax.fori_loop(0, CSZ // 8, inner, carry)

        def body(t, carry):
            j0 = 2 * t
            j1 = j0 + 1
            cpa0 = pltpu.async_copy(st_hbm.at[si_v.at[j0]], ra0, sem0)
            cpb0 = pltpu.async_copy(st_hbm.at[di_v.at[j0]], rb0, sem0)
            cpa1 = pltpu.async_copy(st_hbm.at[si_v.at[j1]], ra1, sem1)
            cpb1 = pltpu.async_copy(st_hbm.at[di_v.at[j1]], rb1, sem1)
            cpa0.wait()
            cpb0.wait()
            carry = accum(ra0, rb0, carry)
            cpa1.wait()
            cpb1.wait()
            return accum(ra1, rb1, carry)

        init = tuple(zero for _ in range(8))
        accs = lax.fori_loop(0, _nchunks(wid) // 2, body, init)
        res_v[0] = accs[0] + accs[1] + accs[2] + accs[3]
        res_v[1] = accs[4] + accs[5] + accs[6] + accs[7]
        pltpu.sync_copy(res_v, out_hbm.at[pl.ds(wid * 2, 2)])

    return k


_sc_deg = _make_sc_deg()
_sc_agg = _make_sc_agg(H)
_sc_edge = _make_sc_edge()


def _tc_a_body(x_ref, w1_ref, h1_ref):
    h1_ref[...] = jnp.dot(x_ref[...], w1_ref[...],
                          preferred_element_type=jnp.float32)


def _tc_a(x, w1):
    return pl.pallas_call(
        _tc_a_body,
        grid=(GRID,),
        in_specs=[
            pl.BlockSpec((BL, DIN), lambda i: (i, 0)),
            pl.BlockSpec((DIN, H), lambda i: (0, 0)),
        ],
        out_specs=pl.BlockSpec((BL, H), lambda i: (i, 0)),
        out_shape=jax.ShapeDtypeStruct((N, H), jnp.float32),
    )(x, w1)


def _tc_b_body(h1_ref, dp_ref, u1_ref, dis_ref):
    deg = dp_ref[0] + dp_ref[1] + 1.0
    dis = lax.rsqrt(deg)[:, 0:1]
    u1_ref[...] = dis * h1_ref[...]
    dis_ref[...] = dis


def _tc_b(h1, degp):
    return pl.pallas_call(
        _tc_b_body,
        grid=(GRID,),
        in_specs=[
            pl.BlockSpec((BL, H), lambda i: (i, 0)),
            pl.BlockSpec((NC, BL, L), lambda i: (0, i, 0)),
        ],
        out_specs=[
            pl.BlockSpec((BL, H), lambda i: (i, 0)),
            pl.BlockSpec((BL, 1), lambda i: (i, 0)),
        ],
        out_shape=[
            jax.ShapeDtypeStruct((N, H), jnp.float32),
            jax.ShapeDtypeStruct((N, 1), jnp.float32),
        ],
    )(h1, degp)


def _tc_c_body(ap_ref, u1_ref, dis_ref, b1_ref, w2_ref, u2_ref):
    d0 = dis_ref[...]
    h = jnp.maximum(
        d0 * (ap_ref[0] + ap_ref[1] + u1_ref[...]) + b1_ref[...], 0.0)
    u2_ref[...] = d0 * jnp.dot(h, w2_ref[...],
                               preferred_element_type=jnp.float32)


def _tc_c(ap, u1, dis, b1r, w2):
    return pl.pallas_call(
        _tc_c_body,
        grid=(GRID,),
        in_specs=[
            pl.BlockSpec((NC, BL, H), lambda i: (0, i, 0)),
            pl.BlockSpec((BL, H), lambda i: (i, 0)),
            pl.BlockSpec((BL, 1), lambda i: (i, 0)),
            pl.BlockSpec((1, H), lambda i: (0, 0)),
            pl.BlockSpec((H, H), lambda i: (0, 0)),
        ],
        out_specs=pl.BlockSpec((BL, H), lambda i: (i, 0)),
        out_shape=jax.ShapeDtypeStruct((N, H), jnp.float32),
    )(ap, u1, dis, b1r, w2)


def _tc_d_body(ap_ref, u2_ref, dis_ref, b2_ref, wp_ref, bp_ref,
               st_ref, ss_ref, h_ref):
    i = pl.program_id(0)
    d0 = dis_ref[...]
    h = jnp.maximum(
        d0 * (ap_ref[0] + ap_ref[1] + u2_ref[...]) + b2_ref[...], 0.0)
    h_ref[...] = h
    sl = jnp.dot(h, wp_ref[...], preferred_element_type=jnp.float32) \
        + bp_ref[...]
    m = jnp.max(sl, axis=1, keepdims=True)
    p = jnp.exp(sl - m)
    s = p / jnp.sum(p, axis=1, keepdims=True)
    st_ref[...] = jnp.concatenate(
        [s, jnp.ones((BL, 1), jnp.float32), jnp.zeros((BL, 5), jnp.float32)],
        axis=1)
    ssb = lax.dot_general(s, s, (((0,), (0,)), ((), ())),
                          preferred_element_type=jnp.float32)

    @pl.when(i == 0)
    def _():
        ss_ref[...] = jnp.zeros_like(ss_ref)

    ss_ref[...] += ssb


def _tc_d(ap, u2, dis, b2r, wp, bpr):
    return pl.pallas_call(
        _tc_d_body,
        grid=(GRID,),
        in_specs=[
            pl.BlockSpec((NC, BL, H), lambda i: (0, i, 0)),
            pl.BlockSpec((BL, H), lambda i: (i, 0)),
            pl.BlockSpec((BL, 1), lambda i: (i, 0)),
            pl.BlockSpec((1, H), lambda i: (0, 0)),
            pl.BlockSpec((H, C), lambda i: (0, 0)),
            pl.BlockSpec((1, C), lambda i: (0, 0)),
        ],
        out_specs=[
            pl.BlockSpec((BL, L), lambda i: (i, 0)),
            pl.BlockSpec((C, C), lambda i: (0, 0)),
            pl.BlockSpec((BL, H), lambda i: (i, 0)),
        ],
        out_shape=[
            jax.ShapeDtypeStruct((N, L), jnp.float32),
            jax.ShapeDtypeStruct((C, C), jnp.float32),
            jax.ShapeDtypeStruct((N, H), jnp.float32),
        ],
    )(ap, u2, dis, b2r, wp, bpr)


def _tc_d2_body(h_ref, wc_ref, bc_ref, lp_ref):
    lo = jnp.dot(h_ref[...], wc_ref[...],
                 preferred_element_type=jnp.float32) + bc_ref[...]
    mm = jnp.max(lo, axis=1, keepdims=True)
    lp_ref[...] = lo - mm - jnp.log(
        jnp.sum(jnp.exp(lo - mm), axis=1, keepdims=True))


def _tc_d2(h, wc, bcr):
    return pl.pallas_call(
        _tc_d2_body,
        grid=(GRID,),
        in_specs=[
            pl.BlockSpec((BL, H), lambda i: (i, 0)),
            pl.BlockSpec((H, NCLS), lambda i: (0, 0)),
            pl.BlockSpec((1, NCLS), lambda i: (0, 0)),
        ],
        out_specs=pl.BlockSpec((BL, NCLS), lambda i: (i, 0)),
        out_shape=jax.ShapeDtypeStruct((N, NCLS), jnp.float32),
    )(h, wc, bcr)


def _tc_f_body(na_ref, da_ref, ss_ref, mc_ref, o_ref):
    num = jnp.sum(na_ref[...]) - float(E)
    den = jnp.sum(da_ref[...]) - float(E)
    mc_ref[...] = jnp.full((1, 1), -(num / den), jnp.float32)
    ssv = ss_ref[...]
    nss = jnp.sqrt(jnp.sum(ssv * ssv))
    eye = (lax.broadcasted_iota(jnp.int32, (C, C), 0)
           == lax.broadcasted_iota(jnp.int32, (C, C), 1)).astype(jnp.float32)
    t = ssv / nss - eye / jnp.sqrt(jnp.float32(C))
    o_ref[...] = jnp.full((1, 1), jnp.sqrt(jnp.sum(t * t)), jnp.float32)


def _tc_f(na, da, ss):
    return pl.pallas_call(
        _tc_f_body,
        out_shape=[
            jax.ShapeDtypeStruct((1, 1), jnp.float32),
            jax.ShapeDtypeStruct((1, 1), jnp.float32),
        ],
    )(na, da, ss)


def kernel(x, edge_index, W1, b1, W2, b2, Wp, bp, Wc, bc):
    ei = jnp.pad(edge_index, ((0, 0), (0, EP - E))).reshape(2, EPC, CSZ)

    degp = _sc_deg(ei).reshape(NC, NPAD, L)
    h1 = _tc_a(x, W1)
    u1, dis = _tc_b(h1, degp)

    a1 = _sc_agg(u1, ei).reshape(NC, NPAD, H)
    u2 = _tc_c(a1, u1, dis, b1.reshape(1, H), W2)

    a2 = _sc_agg(u2, ei).reshape(NC, NPAD, H)
    st, ss, h = _tc_d(a2, u2, dis, b2.reshape(1, H), Wp, bp.reshape(1, C))

    eacc = _sc_edge(st, ei).reshape(NW, 2, L)
    logp = _tc_d2(h, Wc, bc.reshape(1, NCLS))
    mc, o = _tc_f(eacc[:, 0], eacc[:, 1], ss)
    return (logp, jnp.reshape(mc, ()), jnp.reshape(o, ()))


# speedup vs baseline: 1.1716x; 1.0801x over previous
import functools

import jax
import jax.numpy as jnp
from jax import lax
from jax.experimental import pallas as pl
from jax.experimental.pallas import tpu as pltpu
from jax.experimental.pallas import tpu_sc as plsc

N = 10000
DIN = 128
H = 32
C = 10
NCLS = 7
E = 160000

NC = 2
NS = 16
NW = NC * NS
L = 16

NPAD = 10112
RPW = NPAD // NS
CSZ = 128
CH = 40
EPT = CH * CSZ
CH_LAST = (E - (NW - 1) * EPT) // CSZ
EP = NW * EPT
EPC = EP // CSZ

BL = 2000
GRID = N // BL

_MESH = dict(core_axis_name="c", subcore_axis_name="s", num_cores=NC,
             num_subcores=NS)
_SC_PARAMS = pltpu.CompilerParams(use_tc_tiling_on_sc=False)


def _nchunks(wid):
    return jnp.where(wid == NW - 1, CH_LAST, CH)


def _make_sc_deg():
    mesh = plsc.VectorSubcoreMesh(**_MESH)

    @functools.partial(
        pl.kernel,
        out_type=jax.ShapeDtypeStruct((NC * NPAD, L), jnp.float32),
        mesh=mesh,
        compiler_params=_SC_PARAMS,
        scratch_types=[
            pltpu.VMEM((CH, CSZ), jnp.int32),
            pltpu.VMEM((CSZ, L), jnp.float32),
            pltpu.VMEM((RPW, L), jnp.float32),
            pltpu.VMEM_SHARED((NPAD, L), jnp.float32),
            pltpu.SemaphoreType.DMA,
        ],
    )
    def k(ei_hbm, out_hbm, si_v, ones_v, stage_v, acc_sh, sem):
        cid = lax.axis_index("c")
        sid = lax.axis_index("s")
        wid = sid * NC + cid
        pltpu.sync_copy(ei_hbm.at[1, pl.ds(wid * CH, CH)], si_v)
        one = jnp.full((L,), 1.0, jnp.float32)
        zero = jnp.zeros((L,), jnp.float32)

        def f1(i, _):
            ones_v[i, :] = one
            return 0

        lax.fori_loop(0, CSZ, f1, 0)

        def fz(i, _):
            stage_v[i, :] = zero
            return 0

        lax.fori_loop(0, RPW, fz, 0)
        pltpu.sync_copy(stage_v, acc_sh.at[pl.ds(sid * RPW, RPW)])
        plsc.subcore_barrier()
        nchk = _nchunks(wid)

        def fire(j, _):
            pltpu.async_copy(ones_v, acc_sh.at[si_v.at[j]], sem, add=True)
            return 0

        lax.fori_loop(0, nchk, fire, 0)

        def drain(j, _):
            pltpu.make_async_copy(ones_v, acc_sh.at[si_v.at[j]], sem).wait()
            return 0

        lax.fori_loop(0, nchk, drain, 0)
        plsc.subcore_barrier()
        pltpu.sync_copy(
            acc_sh.at[pl.ds(sid * RPW, RPW)],
            out_hbm.at[pl.ds(cid * NPAD + sid * RPW, RPW)],
        )

    return k


def _make_sc_agg(W):
    mesh = plsc.VectorSubcoreMesh(**_MESH)
    NB = 4

    @functools.partial(
        pl.kernel,
        out_type=jax.ShapeDtypeStruct((NC * NPAD, W), jnp.float32),
        mesh=mesh,
        compiler_params=_SC_PARAMS,
        scratch_types=[
            pltpu.VMEM((CH, CSZ), jnp.int32),
            pltpu.VMEM((CH, CSZ), jnp.int32),
        ] + [pltpu.VMEM((CSZ, W), jnp.float32) for _ in range(NB)] + [
            pltpu.VMEM((RPW, W), jnp.float32),
            pltpu.VMEM_SHARED((NPAD, W), jnp.float32),
            pltpu.VMEM_SHARED((N, W), jnp.float32),
        ] + [pltpu.SemaphoreType.DMA for _ in range(2 * NB)],
    )
    def k(table_hbm, ei_hbm, out_hbm, gi_v, si_v, r0, r1, r2, r3,
          stage_v, acc_sh, tab_sh, sg0, sg1, sg2, sg3, ss0, ss1, ss2, ss3):
        cid = lax.axis_index("c")
        sid = lax.axis_index("s")
        wid = sid * NC + cid
        bufs = (r0, r1, r2, r3)
        gsems = (sg0, sg1, sg2, sg3)
        ssems = (ss0, ss1, ss2, ss3)
        TRW = 624
        pltpu.sync_copy(table_hbm.at[pl.ds(sid * TRW, TRW)],
                        tab_sh.at[pl.ds(sid * TRW, TRW)])

        @pl.when(sid == NS - 1)
        def _():
            pltpu.sync_copy(table_hbm.at[pl.ds(NS * TRW, N - NS * TRW)],
                            tab_sh.at[pl.ds(NS * TRW, N - NS * TRW)])

        pltpu.sync_copy(ei_hbm.at[0, pl.ds(wid * CH, CH)], gi_v)
        pltpu.sync_copy(ei_hbm.at[1, pl.ds(wid * CH, CH)], si_v)
        zero = jnp.zeros((L,), jnp.float32)

        def fz(i, _):
            for w0 in range(W // L):
                stage_v[i, pl.ds(w0 * L, L)] = zero
            return 0

        lax.fori_loop(0, RPW, fz, 0)
        pltpu.sync_copy(stage_v, acc_sh.at[pl.ds(sid * RPW, RPW)])
        plsc.subcore_barrier()
        nchk = _nchunks(wid)

        def do_group(j, nb):
            gathers = [
                pltpu.async_copy(tab_sh.at[gi_v.at[j + b]], bufs[b],
                                 gsems[b])
                for b in range(nb)
            ]
            scatters = []
            for b in range(nb):
                gathers[b].wait()
                scatters.append(
                    pltpu.async_copy(bufs[b], acc_sh.at[si_v.at[j + b]],
                                     ssems[b], add=True))
            for b in range(nb):
                scatters[b].wait()

        def body(t, _):
            do_group(NB * t, NB)
            return 0

        nfull = nchk // NB
        lax.fori_loop(0, nfull, body, 0)

        @pl.when(nchk % NB != 0)
        def _():
            do_group(nfull * NB, 2)

        plsc.subcore_barrier()
        pltpu.sync_copy(
            acc_sh.at[pl.ds(sid * RPW, RPW)],
            out_hbm.at[pl.ds(cid * NPAD + sid * RPW, RPW)],
        )

    return k


def _make_sc_edge():
    mesh = plsc.VectorSubcoreMesh(**_MESH)

    @functools.partial(
        pl.kernel,
        out_type=jax.ShapeDtypeStruct((NW * 2, L), jnp.float32),
        mesh=mesh,
        compiler_params=_SC_PARAMS,
        scratch_types=[
            pltpu.VMEM((CH, CSZ), jnp.int32),
            pltpu.VMEM((CH, CSZ), jnp.int32),
            pltpu.VMEM((CSZ, L), jnp.float32),
            pltpu.VMEM((CSZ, L), jnp.float32),
            pltpu.VMEM((CSZ, L), jnp.float32),
            pltpu.VMEM((CSZ, L), jnp.float32),
            pltpu.VMEM((2, L), jnp.float32),
            pltpu.VMEM_SHARED((N, L), jnp.float32),
            pltpu.SemaphoreType.DMA,
            pltpu.SemaphoreType.DMA,
        ],
    )
    def k(st_hbm, ei_hbm, out_hbm, si_v, di_v, ra0, rb0, ra1, rb1,
          res_v, tab_sh, sem0, sem1):
        cid = lax.axis_index("c")
        sid = lax.axis_index("s")
        wid = sid * NC + cid
        TRW = 624
        pltpu.sync_copy(st_hbm.at[pl.ds(sid * TRW, TRW)],
                        tab_sh.at[pl.ds(sid * TRW, TRW)])

        @pl.when(sid == NS - 1)
        def _():
            pltpu.sync_copy(st_hbm.at[pl.ds(NS * TRW, N - NS * TRW)],
                            tab_sh.at[pl.ds(NS * TRW, N - NS * TRW)])

        pltpu.sync_copy(ei_hbm.at[0, pl.ds(wid * CH, CH)], si_v)
        pltpu.sync_copy(ei_hbm.at[1, pl.ds(wid * CH, CH)], di_v)
        plsc.subcore_barrier()
        zero = jnp.zeros((L,), jnp.float32)

        def accum(ra, rb, carry):
            def inner(r, c2):
                accs = list(c2)
                base = r * 8
                for u in range(8):
                    va = ra[base + u]
                    vb = rb[base + u]
                    p = u % 4
                    accs[p] = accs[p] + va * vb
                    accs[4 + p] = accs[4 + p] + va * va
                return tuple(accs)

            return lax.fori_loop(0, CSZ // 8, inner, carry)

        def body(t, carry):
            j0 = 2 * t
            j1 = j0 + 1
            cpa0 = pltpu.async_copy(tab_sh.at[si_v.at[j0]], ra0, sem0)
            cpb0 = pltpu.async_copy(tab_sh.at[di_v.at[j0]], rb0, sem0)
            cpa1 = pltpu.async_copy(tab_sh.at[si_v.at[j1]], ra1, sem1)
            cpb1 = pltpu.async_copy(tab_sh.at[di_v.at[j1]], rb1, sem1)
            cpa0.wait()
            cpb0.wait()
            carry = accum(ra0, rb0, carry)
            cpa1.wait()
            cpb1.wait()
            return accum(ra1, rb1, carry)

        init = tuple(zero for _ in range(8))
        accs = lax.fori_loop(0, _nchunks(wid) // 2, body, init)
        res_v[0] = accs[0] + accs[1] + accs[2] + accs[3]
        res_v[1] = accs[4] + accs[5] + accs[6] + accs[7]
        pltpu.sync_copy(res_v, out_hbm.at[pl.ds(wid * 2, 2)])

    return k


_sc_deg = _make_sc_deg()
_sc_agg = _make_sc_agg(H)
_sc_edge = _make_sc_edge()


def _tc_a_body(x_ref, w1_ref, h1_ref):
    h1_ref[...] = jnp.dot(x_ref[...], w1_ref[...],
                          preferred_element_type=jnp.float32)


def _tc_a(x, w1):
    return pl.pallas_call(
        _tc_a_body,
        grid=(GRID,),
        in_specs=[
            pl.BlockSpec((BL, DIN), lambda i: (i, 0)),
            pl.BlockSpec((DIN, H), lambda i: (0, 0)),
        ],
        out_specs=pl.BlockSpec((BL, H), lambda i: (i, 0)),
        out_shape=jax.ShapeDtypeStruct((N, H), jnp.float32),
    )(x, w1)


def _tc_b_body(h1_ref, dp_ref, u1_ref, dis_ref):
    deg = dp_ref[0] + dp_ref[1] + 1.0
    dis = lax.rsqrt(deg)[:, 0:1]
    u1_ref[...] = dis * h1_ref[...]
    dis_ref[...] = dis


def _tc_b(h1, degp):
    return pl.pallas_call(
        _tc_b_body,
        grid=(GRID,),
        in_specs=[
            pl.BlockSpec((BL, H), lambda i: (i, 0)),
            pl.BlockSpec((NC, BL, L), lambda i: (0, i, 0)),
        ],
        out_specs=[
            pl.BlockSpec((BL, H), lambda i: (i, 0)),
            pl.BlockSpec((BL, 1), lambda i: (i, 0)),
        ],
        out_shape=[
            jax.ShapeDtypeStruct((N, H), jnp.float32),
            jax.ShapeDtypeStruct((N, 1), jnp.float32),
        ],
    )(h1, degp)


def _tc_c_body(ap_ref, u1_ref, dis_ref, b1_ref, w2_ref, u2_ref):
    d0 = dis_ref[...]
    h = jnp.maximum(
        d0 * (ap_ref[0] + ap_ref[1] + u1_ref[...]) + b1_ref[...], 0.0)
    u2_ref[...] = d0 * jnp.dot(h, w2_ref[...],
                               preferred_element_type=jnp.float32)


def _tc_c(ap, u1, dis, b1r, w2):
    return pl.pallas_call(
        _tc_c_body,
        grid=(GRID,),
        in_specs=[
            pl.BlockSpec((NC, BL, H), lambda i: (0, i, 0)),
            pl.BlockSpec((BL, H), lambda i: (i, 0)),
            pl.BlockSpec((BL, 1), lambda i: (i, 0)),
            pl.BlockSpec((1, H), lambda i: (0, 0)),
            pl.BlockSpec((H, H), lambda i: (0, 0)),
        ],
        out_specs=pl.BlockSpec((BL, H), lambda i: (i, 0)),
        out_shape=jax.ShapeDtypeStruct((N, H), jnp.float32),
    )(ap, u1, dis, b1r, w2)


def _tc_d_body(ap_ref, u2_ref, dis_ref, b2_ref, wp_ref, bp_ref,
               st_ref, ss_ref, h_ref):
    i = pl.program_id(0)
    d0 = dis_ref[...]
    h = jnp.maximum(
        d0 * (ap_ref[0] + ap_ref[1] + u2_ref[...]) + b2_ref[...], 0.0)
    h_ref[...] = h
    sl = jnp.dot(h, wp_ref[...], preferred_element_type=jnp.float32) \
        + bp_ref[...]
    m = jnp.max(sl, axis=1, keepdims=True)
    p = jnp.exp(sl - m)
    s = p / jnp.sum(p, axis=1, keepdims=True)
    st_ref[...] = jnp.concatenate(
        [s, jnp.ones((BL, 1), jnp.float32), jnp.zeros((BL, 5), jnp.float32)],
        axis=1)
    ssb = lax.dot_general(s, s, (((0,), (0,)), ((), ())),
                          preferred_element_type=jnp.float32)

    @pl.when(i == 0)
    def _():
        ss_ref[...] = jnp.zeros_like(ss_ref)

    ss_ref[...] += ssb


def _tc_d(ap, u2, dis, b2r, wp, bpr):
    return pl.pallas_call(
        _tc_d_body,
        grid=(GRID,),
        in_specs=[
            pl.BlockSpec((NC, BL, H), lambda i: (0, i, 0)),
            pl.BlockSpec((BL, H), lambda i: (i, 0)),
            pl.BlockSpec((BL, 1), lambda i: (i, 0)),
            pl.BlockSpec((1, H), lambda i: (0, 0)),
            pl.BlockSpec((H, C), lambda i: (0, 0)),
            pl.BlockSpec((1, C), lambda i: (0, 0)),
        ],
        out_specs=[
            pl.BlockSpec((BL, L), lambda i: (i, 0)),
            pl.BlockSpec((C, C), lambda i: (0, 0)),
            pl.BlockSpec((BL, H), lambda i: (i, 0)),
        ],
        out_shape=[
            jax.ShapeDtypeStruct((N, L), jnp.float32),
            jax.ShapeDtypeStruct((C, C), jnp.float32),
            jax.ShapeDtypeStruct((N, H), jnp.float32),
        ],
    )(ap, u2, dis, b2r, wp, bpr)


def _tc_d2_body(h_ref, wc_ref, bc_ref, lp_ref):
    lo = jnp.dot(h_ref[...], wc_ref[...],
                 preferred_element_type=jnp.float32) + bc_ref[...]
    mm = jnp.max(lo, axis=1, keepdims=True)
    lp_ref[...] = lo - mm - jnp.log(
        jnp.sum(jnp.exp(lo - mm), axis=1, keepdims=True))


def _tc_d2(h, wc, bcr):
    return pl.pallas_call(
        _tc_d2_body,
        grid=(GRID,),
        in_specs=[
            pl.BlockSpec((BL, H), lambda i: (i, 0)),
            pl.BlockSpec((H, NCLS), lambda i: (0, 0)),
            pl.BlockSpec((1, NCLS), lambda i: (0, 0)),
        ],
        out_specs=pl.BlockSpec((BL, NCLS), lambda i: (i, 0)),
        out_shape=jax.ShapeDtypeStruct((N, NCLS), jnp.float32),
    )(h, wc, bcr)


def _tc_f_body(na_ref, da_ref, ss_ref, mc_ref, o_ref):
    num = jnp.sum(na_ref[...]) - float(E)
    den = jnp.sum(da_ref[...]) - float(E)
    mc_ref[...] = jnp.full((1, 1), -(num / den), jnp.float32)
    ssv = ss_ref[...]
    nss = jnp.sqrt(jnp.sum(ssv * ssv))
    eye = (lax.broadcasted_iota(jnp.int32, (C, C), 0)
           == lax.broadcasted_iota(jnp.int32, (C, C), 1)).astype(jnp.float32)
    t = ssv / nss - eye / jnp.sqrt(jnp.float32(C))
    o_ref[...] = jnp.full((1, 1), jnp.sqrt(jnp.sum(t * t)), jnp.float32)


def _tc_f(na, da, ss):
    return pl.pallas_call(
        _tc_f_body,
        out_shape=[
            jax.ShapeDtypeStruct((1, 1), jnp.float32),
            jax.ShapeDtypeStruct((1, 1), jnp.float32),
        ],
    )(na, da, ss)


def kernel(x, edge_index, W1, b1, W2, b2, Wp, bp, Wc, bc):
    ei = jnp.pad(edge_index, ((0, 0), (0, EP - E))).reshape(2, EPC, CSZ)

    degp = _sc_deg(ei).reshape(NC, NPAD, L)
    h1 = _tc_a(x, W1)
    u1, dis = _tc_b(h1, degp)

    a1 = _sc_agg(u1, ei).reshape(NC, NPAD, H)
    u2 = _tc_c(a1, u1, dis, b1.reshape(1, H), W2)

    a2 = _sc_agg(u2, ei).reshape(NC, NPAD, H)
    st, ss, h = _tc_d(a2, u2, dis, b2.reshape(1, H), Wp, bp.reshape(1, C))

    eacc = _sc_edge(st, ei).reshape(NW, 2, L)
    logp = _tc_d2(h, Wc, bc.reshape(1, NCLS))
    mc, o = _tc_f(eacc[:, 0], eacc[:, 1], ss)
    return (logp, jnp.reshape(mc, ()), jnp.reshape(o, ()))


# speedup vs baseline: 1.1720x; 1.0003x over previous
import functools

import jax
import jax.numpy as jnp
from jax import lax
from jax.experimental import pallas as pl
from jax.experimental.pallas import tpu as pltpu
from jax.experimental.pallas import tpu_sc as plsc

N = 10000
DIN = 128
H = 32
C = 10
NCLS = 7
E = 160000

NC = 2
NS = 16
NW = NC * NS
L = 16

NPAD = 10112
RPW = NPAD // NS
CSZ = 128
CH = 40
EPT = CH * CSZ
CH_LAST = (E - (NW - 1) * EPT) // CSZ
EP = NW * EPT
EPC = EP // CSZ

BL = 2000
GRID = N // BL

_MESH = dict(core_axis_name="c", subcore_axis_name="s", num_cores=NC,
             num_subcores=NS)
_SC_PARAMS = pltpu.CompilerParams(use_tc_tiling_on_sc=False)


def _nchunks(wid):
    return jnp.where(wid == NW - 1, CH_LAST, CH)


def _make_sc_deg():
    mesh = plsc.VectorSubcoreMesh(**_MESH)

    @functools.partial(
        pl.kernel,
        out_type=jax.ShapeDtypeStruct((NC * NPAD, L), jnp.float32),
        mesh=mesh,
        compiler_params=_SC_PARAMS,
        scratch_types=[
            pltpu.VMEM((CH, CSZ), jnp.int32),
            pltpu.VMEM((CSZ, L), jnp.float32),
            pltpu.VMEM((RPW, L), jnp.float32),
            pltpu.VMEM_SHARED((NPAD, L), jnp.float32),
            pltpu.SemaphoreType.DMA,
        ],
    )
    def k(ei_hbm, out_hbm, si_v, ones_v, stage_v, acc_sh, sem):
        cid = lax.axis_index("c")
        sid = lax.axis_index("s")
        wid = sid * NC + cid
        pltpu.sync_copy(ei_hbm.at[1, pl.ds(wid * CH, CH)], si_v)
        one = jnp.full((L,), 1.0, jnp.float32)
        zero = jnp.zeros((L,), jnp.float32)

        def f1(i, _):
            ones_v[i, :] = one
            return 0

        lax.fori_loop(0, CSZ, f1, 0)

        def fz(i, _):
            stage_v[i, :] = zero
            return 0

        lax.fori_loop(0, RPW, fz, 0)
        pltpu.sync_copy(stage_v, acc_sh.at[pl.ds(sid * RPW, RPW)])
        plsc.subcore_barrier()
        nchk = _nchunks(wid)

        def fire(j, _):
            pltpu.async_copy(ones_v, acc_sh.at[si_v.at[j]], sem, add=True)
            return 0

        lax.fori_loop(0, nchk, fire, 0)

        def drain(j, _):
            pltpu.make_async_copy(ones_v, acc_sh.at[si_v.at[j]], sem).wait()
            return 0

        lax.fori_loop(0, nchk, drain, 0)
        plsc.subcore_barrier()
        pltpu.sync_copy(
            acc_sh.at[pl.ds(sid * RPW, RPW)],
            out_hbm.at[pl.ds(cid * NPAD + sid * RPW, RPW)],
        )

    return k


def _make_sc_agg(W):
    mesh = plsc.VectorSubcoreMesh(**_MESH)
    NB = 4

    @functools.partial(
        pl.kernel,
        out_type=jax.ShapeDtypeStruct((NC * NPAD, W), jnp.float32),
        mesh=mesh,
        compiler_params=_SC_PARAMS,
        scratch_types=[
            pltpu.VMEM((CH, CSZ), jnp.int32),
            pltpu.VMEM((CH, CSZ), jnp.int32),
        ] + [pltpu.VMEM((CSZ, W), jnp.float32) for _ in range(NB)] + [
            pltpu.VMEM((RPW, W), jnp.float32),
            pltpu.VMEM_SHARED((NPAD, W), jnp.float32),
            pltpu.VMEM_SHARED((N, W), jnp.float32),
        ] + [pltpu.SemaphoreType.DMA for _ in range(2 * NB)],
    )
    def k(table_hbm, ei_hbm, out_hbm, gi_v, si_v, r0, r1, r2, r3,
          stage_v, acc_sh, tab_sh, sg0, sg1, sg2, sg3, ss0, ss1, ss2, ss3):
        cid = lax.axis_index("c")
        sid = lax.axis_index("s")
        wid = sid * NC + cid
        bufs = (r0, r1, r2, r3)
        gsems = (sg0, sg1, sg2, sg3)
        ssems = (ss0, ss1, ss2, ss3)
        TRW = 624
        pltpu.sync_copy(table_hbm.at[pl.ds(sid * TRW, TRW)],
                        tab_sh.at[pl.ds(sid * TRW, TRW)])

        @pl.when(sid == NS - 1)
        def _():
            pltpu.sync_copy(table_hbm.at[pl.ds(NS * TRW, N - NS * TRW)],
                            tab_sh.at[pl.ds(NS * TRW, N - NS * TRW)])

        pltpu.sync_copy(ei_hbm.at[0, pl.ds(wid * CH, CH)], gi_v)
        pltpu.sync_copy(ei_hbm.at[1, pl.ds(wid * CH, CH)], si_v)
        zero = jnp.zeros((L,), jnp.float32)

        def fz(i, _):
            for w0 in range(W // L):
                stage_v[i, pl.ds(w0 * L, L)] = zero
            return 0

        lax.fori_loop(0, RPW, fz, 0)
        pltpu.sync_copy(stage_v, acc_sh.at[pl.ds(sid * RPW, RPW)])
        plsc.subcore_barrier()
        nchk = _nchunks(wid)

        def do_group(j, nb):
            gathers = [
                pltpu.async_copy(tab_sh.at[gi_v.at[j + b]], bufs[b],
                                 gsems[b])
                for b in range(nb)
            ]
            scatters = []
            for b in range(nb):
                gathers[b].wait()
                scatters.append(
                    pltpu.async_copy(bufs[b], acc_sh.at[si_v.at[j + b]],
                                     ssems[b], add=True))
            for b in range(nb):
                scatters[b].wait()

        def body(t, _):
            do_group(NB * t, NB)
            return 0

        nfull = nchk // NB
        lax.fori_loop(0, nfull, body, 0)

        @pl.when(nchk % NB != 0)
        def _():
            do_group(nfull * NB, 2)

        plsc.subcore_barrier()
        pltpu.sync_copy(
            acc_sh.at[pl.ds(sid * RPW, RPW)],
            out_hbm.at[pl.ds(cid * NPAD + sid * RPW, RPW)],
        )

    return k


def _make_sc_edge():
    mesh = plsc.VectorSubcoreMesh(**_MESH)

    @functools.partial(
        pl.kernel,
        out_type=jax.ShapeDtypeStruct((NW * 2, L), jnp.float32),
        mesh=mesh,
        compiler_params=_SC_PARAMS,
        scratch_types=[
            pltpu.VMEM((CH, CSZ), jnp.int32),
            pltpu.VMEM((CH, CSZ), jnp.int32),
            pltpu.VMEM((CSZ, L), jnp.float32),
            pltpu.VMEM((CSZ, L), jnp.float32),
            pltpu.VMEM((CSZ, L), jnp.float32),
            pltpu.VMEM((CSZ, L), jnp.float32),
            pltpu.VMEM((2, L), jnp.float32),
            pltpu.VMEM_SHARED((N, L), jnp.float32),
            pltpu.SemaphoreType.DMA,
            pltpu.SemaphoreType.DMA,
        ],
    )
    def k(st_hbm, ei_hbm, out_hbm, si_v, di_v, ra0, rb0, ra1, rb1,
          res_v, tab_sh, sem0, sem1):
        cid = lax.axis_index("c")
        sid = lax.axis_index("s")
        wid = sid * NC + cid
        TRW = 624
        pltpu.sync_copy(st_hbm.at[pl.ds(sid * TRW, TRW)],
                        tab_sh.at[pl.ds(sid * TRW, TRW)])

        @pl.when(sid == NS - 1)
        def _():
            pltpu.sync_copy(st_hbm.at[pl.ds(NS * TRW, N - NS * TRW)],
                            tab_sh.at[pl.ds(NS * TRW, N - NS * TRW)])

        pltpu.sync_copy(ei_hbm.at[0, pl.ds(wid * CH, CH)], si_v)
        pltpu.sync_copy(ei_hbm.at[1, pl.ds(wid * CH, CH)], di_v)
        plsc.subcore_barrier()
        zero = jnp.zeros((L,), jnp.float32)

        def accum(ra, rb, carry):
            def inner(r, c2):
                accs = list(c2)
                base = r * 16
                for u in range(16):
                    va = ra[base + u]
                    vb = rb[base + u]
                    p = u % 4
                    accs[p] = accs[p] + va * vb
                    accs[4 + p] = accs[4 + p] + va * va
                return tuple(accs)

            return lax.fori_loop(0, CSZ // 16, inner, carry)

        def body(t, carry):
            j0 = 2 * t
            j1 = j0 + 1
            cpa0 = pltpu.async_copy(tab_sh.at[si_v.at[j0]], ra0, sem0)
            cpb0 = pltpu.async_copy(tab_sh.at[di_v.at[j0]], rb0, sem0)
            cpa1 = pltpu.async_copy(tab_sh.at[si_v.at[j1]], ra1, sem1)
            cpb1 = pltpu.async_copy(tab_sh.at[di_v.at[j1]], rb1, sem1)
            cpa0.wait()
            cpb0.wait()
            carry = accum(ra0, rb0, carry)
            cpa1.wait()
            cpb1.wait()
            return accum(ra1, rb1, carry)

        init = tuple(zero for _ in range(8))
        accs = lax.fori_loop(0, _nchunks(wid) // 2, body, init)
        res_v[0] = accs[0] + accs[1] + accs[2] + accs[3]
        res_v[1] = accs[4] + accs[5] + accs[6] + accs[7]
        pltpu.sync_copy(res_v, out_hbm.at[pl.ds(wid * 2, 2)])

    return k


_sc_deg = _make_sc_deg()
_sc_agg = _make_sc_agg(H)
_sc_edge = _make_sc_edge()


def _tc_a_body(x_ref, w1_ref, h1_ref):
    h1_ref[...] = jnp.dot(x_ref[...], w1_ref[...],
                          preferred_element_type=jnp.float32)


def _tc_a(x, w1):
    return pl.pallas_call(
        _tc_a_body,
        grid=(GRID,),
        in_specs=[
            pl.BlockSpec((BL, DIN), lambda i: (i, 0)),
            pl.BlockSpec((DIN, H), lambda i: (0, 0)),
        ],
        out_specs=pl.BlockSpec((BL, H), lambda i: (i, 0)),
        out_shape=jax.ShapeDtypeStruct((N, H), jnp.float32),
    )(x, w1)


def _tc_b_body(h1_ref, dp_ref, u1_ref, dis_ref):
    deg = dp_ref[0] + dp_ref[1] + 1.0
    dis = lax.rsqrt(deg)[:, 0:1]
    u1_ref[...] = dis * h1_ref[...]
    dis_ref[...] = dis


def _tc_b(h1, degp):
    return pl.pallas_call(
        _tc_b_body,
        grid=(GRID,),
        in_specs=[
            pl.BlockSpec((BL, H), lambda i: (i, 0)),
            pl.BlockSpec((NC, BL, L), lambda i: (0, i, 0)),
        ],
        out_specs=[
            pl.BlockSpec((BL, H), lambda i: (i, 0)),
            pl.BlockSpec((BL, 1), lambda i: (i, 0)),
        ],
        out_shape=[
            jax.ShapeDtypeStruct((N, H), jnp.float32),
            jax.ShapeDtypeStruct((N, 1), jnp.float32),
        ],
    )(h1, degp)


def _tc_c_body(ap_ref, u1_ref, dis_ref, b1_ref, w2_ref, u2_ref):
    d0 = dis_ref[...]
    h = jnp.maximum(
        d0 * (ap_ref[0] + ap_ref[1] + u1_ref[...]) + b1_ref[...], 0.0)
    u2_ref[...] = d0 * jnp.dot(h, w2_ref[...],
                               preferred_element_type=jnp.float32)


def _tc_c(ap, u1, dis, b1r, w2):
    return pl.pallas_call(
        _tc_c_body,
        grid=(GRID,),
        in_specs=[
            pl.BlockSpec((NC, BL, H), lambda i: (0, i, 0)),
            pl.BlockSpec((BL, H), lambda i: (i, 0)),
            pl.BlockSpec((BL, 1), lambda i: (i, 0)),
            pl.BlockSpec((1, H), lambda i: (0, 0)),
            pl.BlockSpec((H, H), lambda i: (0, 0)),
        ],
        out_specs=pl.BlockSpec((BL, H), lambda i: (i, 0)),
        out_shape=jax.ShapeDtypeStruct((N, H), jnp.float32),
    )(ap, u1, dis, b1r, w2)


def _tc_d_body(ap_ref, u2_ref, dis_ref, b2_ref, wp_ref, bp_ref,
               st_ref, ss_ref, h_ref):
    i = pl.program_id(0)
    d0 = dis_ref[...]
    h = jnp.maximum(
        d0 * (ap_ref[0] + ap_ref[1] + u2_ref[...]) + b2_ref[...], 0.0)
    h_ref[...] = h
    sl = jnp.dot(h, wp_ref[...], preferred_element_type=jnp.float32) \
        + bp_ref[...]
    m = jnp.max(sl, axis=1, keepdims=True)
    p = jnp.exp(sl - m)
    s = p / jnp.sum(p, axis=1, keepdims=True)
    st_ref[...] = jnp.concatenate(
        [s, jnp.ones((BL, 1), jnp.float32), jnp.zeros((BL, 5), jnp.float32)],
        axis=1)
    ssb = lax.dot_general(s, s, (((0,), (0,)), ((), ())),
                          preferred_element_type=jnp.float32)

    @pl.when(i == 0)
    def _():
        ss_ref[...] = jnp.zeros_like(ss_ref)

    ss_ref[...] += ssb


def _tc_d(ap, u2, dis, b2r, wp, bpr):
    return pl.pallas_call(
        _tc_d_body,
        grid=(GRID,),
        in_specs=[
            pl.BlockSpec((NC, BL, H), lambda i: (0, i, 0)),
            pl.BlockSpec((BL, H), lambda i: (i, 0)),
            pl.BlockSpec((BL, 1), lambda i: (i, 0)),
            pl.BlockSpec((1, H), lambda i: (0, 0)),
            pl.BlockSpec((H, C), lambda i: (0, 0)),
            pl.BlockSpec((1, C), lambda i: (0, 0)),
        ],
        out_specs=[
            pl.BlockSpec((BL, L), lambda i: (i, 0)),
            pl.BlockSpec((C, C), lambda i: (0, 0)),
            pl.BlockSpec((BL, H), lambda i: (i, 0)),
        ],
        out_shape=[
            jax.ShapeDtypeStruct((N, L), jnp.float32),
            jax.ShapeDtypeStruct((C, C), jnp.float32),
            jax.ShapeDtypeStruct((N, H), jnp.float32),
        ],
    )(ap, u2, dis, b2r, wp, bpr)


def _tc_d2_body(h_ref, wc_ref, bc_ref, lp_ref):
    lo = jnp.dot(h_ref[...], wc_ref[...],
                 preferred_element_type=jnp.float32) + bc_ref[...]
    mm = jnp.max(lo, axis=1, keepdims=True)
    lp_ref[...] = lo - mm - jnp.log(
        jnp.sum(jnp.exp(lo - mm), axis=1, keepdims=True))


def _tc_d2(h, wc, bcr):
    return pl.pallas_call(
        _tc_d2_body,
        grid=(GRID,),
        in_specs=[
            pl.BlockSpec((BL, H), lambda i: (i, 0)),
            pl.BlockSpec((H, NCLS), lambda i: (0, 0)),
            pl.BlockSpec((1, NCLS), lambda i: (0, 0)),
        ],
        out_specs=pl.BlockSpec((BL, NCLS), lambda i: (i, 0)),
        out_shape=jax.ShapeDtypeStruct((N, NCLS), jnp.float32),
    )(h, wc, bcr)


def _tc_f_body(na_ref, da_ref, ss_ref, mc_ref, o_ref):
    num = jnp.sum(na_ref[...]) - float(E)
    den = jnp.sum(da_ref[...]) - float(E)
    mc_ref[...] = jnp.full((1, 1), -(num / den), jnp.float32)
    ssv = ss_ref[...]
    nss = jnp.sqrt(jnp.sum(ssv * ssv))
    eye = (lax.broadcasted_iota(jnp.int32, (C, C), 0)
           == lax.broadcasted_iota(jnp.int32, (C, C), 1)).astype(jnp.float32)
    t = ssv / nss - eye / jnp.sqrt(jnp.float32(C))
    o_ref[...] = jnp.full((1, 1), jnp.sqrt(jnp.sum(t * t)), jnp.float32)


def _tc_f(na, da, ss):
    return pl.pallas_call(
        _tc_f_body,
        out_shape=[
            jax.ShapeDtypeStruct((1, 1), jnp.float32),
            jax.ShapeDtypeStruct((1, 1), jnp.float32),
        ],
    )(na, da, ss)


def kernel(x, edge_index, W1, b1, W2, b2, Wp, bp, Wc, bc):
    ei = jnp.pad(edge_index, ((0, 0), (0, EP - E))).reshape(2, EPC, CSZ)

    degp = _sc_deg(ei).reshape(NC, NPAD, L)
    h1 = _tc_a(x, W1)
    u1, dis = _tc_b(h1, degp)

    a1 = _sc_agg(u1, ei).reshape(NC, NPAD, H)
    u2 = _tc_c(a1, u1, dis, b1.reshape(1, H), W2)

    a2 = _sc_agg(u2, ei).reshape(NC, NPAD, H)
    st, ss, h = _tc_d(a2, u2, dis, b2.reshape(1, H), Wp, bp.reshape(1, C))

    eacc = _sc_edge(st, ei).reshape(NW, 2, L)
    logp = _tc_d2(h, Wc, bc.reshape(1, NCLS))
    mc, o = _tc_f(eacc[:, 0], eacc[:, 1], ss)
    return (logp, jnp.reshape(mc, ()), jnp.reshape(o, ()))
